# baseline (speedup 1.0000x reference)
import jax
import jax.numpy as jnp
from jax import lax
from jax.experimental import pallas as pl
from jax.experimental.pallas import tpu as pltpu
from jax.experimental.pallas import tpu_sc as plsc

N = 10000
E = 320000
D = 128

NC = 2
NS = 16
NW = NC * NS

EPW1 = E // NW
CH1 = 400
NCH1 = EPW1 // CH1

NPC = N // NC
ACC_ROWS = 5008
EPW = E // NS
CHUNK = 320
NPAIR = 31
TAIL = EPW - 2 * NPAIR * CHUNK
RPT = 312


def _proj_body(x_ref, wt_ref, a8_ref, wh_ref, s12_ref):
    wh = jnp.dot(x_ref[...], wt_ref[...], preferred_element_type=jnp.float32)
    wh_ref[...] = wh
    s12_ref[...] = jnp.dot(wh, a8_ref[...], preferred_element_type=jnp.float32)


def _proj(x, wt, a8):
    B = 1000
    return pl.pallas_call(
        _proj_body,
        grid=(N // B,),
        in_specs=[
            pl.BlockSpec((B, D), lambda i: (i, 0)),
            pl.BlockSpec((D, D), lambda i: (0, 0)),
            pl.BlockSpec((D, D), lambda i: (0, 0)),
        ],
        out_specs=[
            pl.BlockSpec((B, D), lambda i: (i, 0)),
            pl.BlockSpec((B, D), lambda i: (i, 0)),
        ],
        out_shape=[
            jax.ShapeDtypeStruct((N, D), jnp.float32),
            jax.ShapeDtypeStruct((N, D), jnp.float32),
        ],
    )(x, wt, a8)


def _wpass_body(src_hbm, dst_hbm, s1_hbm, s2_hbm,
                w_hbm, alpha_hbm,
                s1_v, s2_v, sv, dv, wv, alpha_acc, sem):
    c = lax.axis_index("c")
    s = lax.axis_index("s")
    wid = c * NS + s

    pltpu.sync_copy(s1_hbm, s1_v)
    pltpu.sync_copy(s2_hbm, s2_v)

    def _zw(i, carry):
        wv[pl.ds(i * 16, 16)] = jnp.zeros((16,), jnp.float32)
        return carry

    lax.fori_loop(0, CH1 // 16, _zw, 0)

    @pl.when(s == 0)
    def _():
        def _za(k, carry):
            pltpu.sync_copy(wv, alpha_acc.at[pl.ds(k * CH1, CH1)])
            return carry
        lax.fori_loop(0, N // CH1, _za, 0)

    plsc.subcore_barrier()

    def _chunk(k, carry):
        off = wid * EPW1 + k * CH1
        pltpu.sync_copy(src_hbm.at[pl.ds(off, CH1)], sv)
        pltpu.sync_copy(dst_hbm.at[pl.ds(off, CH1)], dv)

        def _w16(j, carry2):
            sidx = sv[pl.ds(j * 16, 16)]
            didx = dv[pl.ds(j * 16, 16)]
            e = (plsc.load_gather(s1_v, [sidx])
                 + plsc.load_gather(s2_v, [didx]))
            e = jnp.where(e > 0, e, 0.2 * e)
            wv[pl.ds(j * 16, 16)] = jnp.exp(e)
            return carry2

        lax.fori_loop(0, CH1 // 16, _w16, 0)

        pltpu.sync_copy(wv, w_hbm.at[pl.ds(off, CH1)])
        pltpu.async_copy(wv, alpha_acc.at[dv], sem, add=True).wait()
        return carry

    lax.fori_loop(0, NCH1, _chunk, 0)

    plsc.subcore_barrier()

    @pl.when(s == 0)
    def _():
        pltpu.sync_copy(alpha_acc, alpha_hbm.at[c, 0])


_wpass = pl.kernel(
    _wpass_body,
    out_type=(
        jax.ShapeDtypeStruct((E,), jnp.float32),
        jax.ShapeDtypeStruct((NC, 8, N), jnp.float32),
    ),
    mesh=plsc.VectorSubcoreMesh(core_axis_name="c", subcore_axis_name="s"),
    compiler_params=pltpu.CompilerParams(needs_layout_passes=False),
    scratch_types=[
        pltpu.VMEM((N,), jnp.float32),
        pltpu.VMEM((N,), jnp.float32),
        pltpu.VMEM((CH1,), jnp.int32),
        pltpu.VMEM((CH1,), jnp.int32),
        pltpu.VMEM((CH1,), jnp.float32),
        pltpu.VMEM_SHARED((N,), jnp.float32),
        pltpu.SemaphoreType.DMA,
    ],
)


def _edge_body(src_hbm, dst_hbm, w_hbm, wh_hbm,
               out_hbm,
               srcv0, srcv1, dstv0, dstv1, wv0, wv1, dstv20, dstv21,
               wm_v, rows0, rows1, srcvt, dstvt, wvt, dstv2t,
               out_acc, sem_g, sem_i, sem_s):
    srcv = (srcv0, srcv1)
    dstv = (dstv0, dstv1)
    wv = (wv0, wv1)
    dstv2 = (dstv20, dstv21)
    rows = (rows0, rows1)
    c = lax.axis_index("c")
    s = lax.axis_index("s")
    base = c * NPC

    def _zr(i, carry):
        for f in range(D // 16):
            rows0[i, pl.ds(f * 16, 16)] = jnp.zeros((16,), jnp.float32)
        return carry

    lax.fori_loop(0, CHUNK, _zr, 0)

    pltpu.sync_copy(rows0.at[pl.ds(0, RPT)],
                    out_acc.at[pl.ds(s * RPT, RPT)])

    @pl.when(s == NS - 1)
    def _():
        rem = ACC_ROWS - NS * RPT
        pltpu.sync_copy(rows0.at[pl.ds(0, rem)],
                        out_acc.at[pl.ds(NS * RPT, rem)])

    plsc.subcore_barrier()

    ebase = s * EPW

    pltpu.sync_copy(src_hbm.at[pl.ds(ebase, CHUNK)], srcv0)
    pltpu.sync_copy(dst_hbm.at[pl.ds(ebase, CHUNK)], dstv0)
    pltpu.sync_copy(w_hbm.at[pl.ds(ebase, CHUNK)], wv0)
    pltpu.async_copy(wh_hbm.at[srcv0], rows0, sem_g)

    def _pair(kk, carry):
        for b in range(2):
            nb = 1 - b
            k = kk * 2 + b
            have_next = (kk < NPAIR - 1) if b == 1 else True
            off_next = ebase + (k + 1) * CHUNK

            def _prefetch():
                pltpu.async_copy(src_hbm.at[pl.ds(off_next, CHUNK)],
                                 srcv[nb], sem_i)
                pltpu.async_copy(dst_hbm.at[pl.ds(off_next, CHUNK)],
                                 dstv[nb], sem_i)
                pltpu.async_copy(w_hbm.at[pl.ds(off_next, CHUNK)],
                                 wv[nb], sem_i)

            if b == 0:
                _prefetch()
            else:
                @pl.when(have_next)
                def _():
                    _prefetch()

            def _w16(j, carry2):
                didx = dstv[b][pl.ds(j * 16, 16)]
                w = wv[b][pl.ds(j * 16, 16)]
                lo = didx - base
                valid = jnp.logical_and(lo >= 0, lo < NPC)
                wm_v[pl.ds(j * 16, 16)] = jnp.where(valid, w, 0.0)
                dstv2[b][pl.ds(j * 16, 16)] = jnp.where(valid, lo, 0)
                return carry2

            lax.fori_loop(0, CHUNK // 16, _w16, 0)

            def _launch_next():
                pltpu.make_async_copy(src_hbm.at[pl.ds(off_next, CHUNK)],
                                      srcv[nb], sem_i).wait()
                pltpu.make_async_copy(dst_hbm.at[pl.ds(off_next, CHUNK)],
                                      dstv[nb], sem_i).wait()
                pltpu.make_async_copy(w_hbm.at[pl.ds(off_next, CHUNK)],
                                      wv[nb], sem_i).wait()
                pltpu.async_copy(wh_hbm.at[srcv[nb]], rows[nb], sem_g)

            def _wait_prev_scatter():
                pltpu.make_async_copy(rows[nb], out_acc.at[dstv2[nb]],
                                      sem_s).wait()

            if b == 0:
                @pl.when(kk >= 1)
                def _():
                    _wait_prev_scatter()
                _launch_next()
            else:
                @pl.when(have_next)
                def _():
                    _wait_prev_scatter()
                    _launch_next()

            pltpu.make_async_copy(wh_hbm.at[srcv[b]], rows[b],
                                  sem_g).wait()

            def _sc16(j, carry2):
                w = wm_v[pl.ds(j * 16, 16)]
                for l in range(16):
                    wl = w.at[jnp.full((16,), l, jnp.int32)].get(
                        mode="promise_in_bounds")
                    r = j * 16 + l
                    for f in range(D // 16):
                        rows[b][r, pl.ds(f * 16, 16)] = (
                            rows[b][r, pl.ds(f * 16, 16)] * wl)
                return carry2

            lax.fori_loop(0, CHUNK // 16, _sc16, 0)

            pltpu.async_copy(rows[b], out_acc.at[dstv2[b]], sem_s,
                             add=True)
        return carry

    lax.fori_loop(0, NPAIR, _pair, 0)

    offt = ebase + 2 * NPAIR * CHUNK
    pltpu.sync_copy(src_hbm.at[pl.ds(offt, TAIL)], srcvt)
    pltpu.sync_copy(dst_hbm.at[pl.ds(offt, TAIL)], dstvt)
    pltpu.sync_copy(w_hbm.at[pl.ds(offt, TAIL)], wvt)

    pltpu.make_async_copy(rows0, out_acc.at[dstv20], sem_s).wait()
    pltpu.async_copy(wh_hbm.at[srcvt], rows0.at[pl.ds(0, TAIL)], sem_g)

    def _w16t(j, carry2):
        didx = dstvt[pl.ds(j * 16, 16)]
        w = wvt[pl.ds(j * 16, 16)]
        lo = didx - base
        valid = jnp.logical_and(lo >= 0, lo < NPC)
        wm_v[pl.ds(j * 16, 16)] = jnp.where(valid, w, 0.0)
        dstv2t[pl.ds(j * 16, 16)] = jnp.where(valid, lo, 0)
        return carry2

    lax.fori_loop(0, TAIL // 16, _w16t, 0)

    pltpu.make_async_copy(wh_hbm.at[srcvt], rows0.at[pl.ds(0, TAIL)],
                          sem_g).wait()

    def _sc16t(j, carry2):
        w = wm_v[pl.ds(j * 16, 16)]
        for l in range(16):
            wl = w.at[jnp.full((16,), l, jnp.int32)].get(
                mode="promise_in_bounds")
            r = j * 16 + l
            for f in range(D // 16):
                rows0[r, pl.ds(f * 16, 16)] = (
                    rows0[r, pl.ds(f * 16, 16)] * wl)
        return carry2

    lax.fori_loop(0, TAIL // 16, _sc16t, 0)

    pltpu.async_copy(rows0.at[pl.ds(0, TAIL)], out_acc.at[dstv2t], sem_s,
                     add=True).wait()
    pltpu.make_async_copy(rows1, out_acc.at[dstv21], sem_s).wait()

    plsc.subcore_barrier()

    pltpu.sync_copy(out_acc.at[pl.ds(s * RPT, RPT)],
                    out_hbm.at[c, pl.ds(s * RPT, RPT)])

    @pl.when(s == NS - 1)
    def _():
        rem = ACC_ROWS - NS * RPT
        pltpu.sync_copy(out_acc.at[pl.ds(NS * RPT, rem)],
                        out_hbm.at[c, pl.ds(NS * RPT, rem)])


_edge_kernel = pl.kernel(
    _edge_body,
    out_type=jax.ShapeDtypeStruct((NC, ACC_ROWS, D), jnp.float32),
    mesh=plsc.VectorSubcoreMesh(core_axis_name="c", subcore_axis_name="s"),
    compiler_params=pltpu.CompilerParams(needs_layout_passes=False),
    scratch_types=[
        pltpu.VMEM((CHUNK,), jnp.int32),
        pltpu.VMEM((CHUNK,), jnp.int32),
        pltpu.VMEM((CHUNK,), jnp.int32),
        pltpu.VMEM((CHUNK,), jnp.int32),
        pltpu.VMEM((CHUNK,), jnp.float32),
        pltpu.VMEM((CHUNK,), jnp.float32),
        pltpu.VMEM((CHUNK,), jnp.int32),
        pltpu.VMEM((CHUNK,), jnp.int32),
        pltpu.VMEM((CHUNK,), jnp.float32),
        pltpu.VMEM((CHUNK, D), jnp.float32),
        pltpu.VMEM((CHUNK, D), jnp.float32),
        pltpu.VMEM((TAIL,), jnp.int32),
        pltpu.VMEM((TAIL,), jnp.int32),
        pltpu.VMEM((TAIL,), jnp.float32),
        pltpu.VMEM((TAIL,), jnp.int32),
        pltpu.VMEM_SHARED((ACC_ROWS, D), jnp.float32),
        pltpu.SemaphoreType.DMA,
        pltpu.SemaphoreType.DMA,
        pltpu.SemaphoreType.DMA,
    ],
)


def _final_body(p_ref, at_ref, o_ref):
    a = jnp.maximum(at_ref[:, 0] + at_ref[:, 1], 1e-10)
    o_ref[...] = p_ref[...] / a[:, None]


def _final(out_cat, alpha_t):
    B = 1000
    return pl.pallas_call(
        _final_body,
        grid=(N // B,),
        in_specs=[
            pl.BlockSpec((B, D), lambda i: (i, 0)),
            pl.BlockSpec((B, 8), lambda i: (i, 0)),
        ],
        out_specs=pl.BlockSpec((B, D), lambda i: (i, 0)),
        out_shape=jax.ShapeDtypeStruct((N, D), jnp.float32),
    )(out_cat, alpha_t)


def kernel(x, edge_index, W, attn_w):
    src = edge_index[0].astype(jnp.int32)
    dst = edge_index[1].astype(jnp.int32)
    wt = W.T
    a1 = attn_w[0, :D]
    a2 = attn_w[0, D:]
    a8 = jnp.zeros((D, D), jnp.float32).at[:, 0].set(a1).at[:, 1].set(a2)
    wh, s12 = _proj(x, wt, a8)
    s1 = s12[:, 0]
    s2 = s12[:, 1]
    w_edges, alpha_parts = _wpass(src, dst, s1, s2)
    out_parts = _edge_kernel(src, dst, w_edges, wh)
    out_cat = jnp.concatenate(
        [out_parts[0, :NPC], out_parts[1, :NPC]], axis=0)
    ap = alpha_parts[:, 0, :]
    alpha_t = jnp.pad(ap.T, ((0, 0), (0, 6)))
    return _final(out_cat, alpha_t)


# speedup vs baseline: 1.4322x; 1.4322x over previous
import jax
import jax.numpy as jnp
from jax import lax
from jax.experimental import pallas as pl
from jax.experimental.pallas import tpu as pltpu
from jax.experimental.pallas import tpu_sc as plsc

N = 10000
E = 320000
D = 128

NC = 2
NS = 16
NW = NC * NS

EPW1 = E // NW
CH1 = 400
NCH1 = EPW1 // CH1

NPC = N // NC
ACC_ROWS = 5008
EPW = E // NS
CHUNK = 320
NPAIR = 31
TAIL = EPW - 2 * NPAIR * CHUNK
RPT = 312


def _proj_body(x_ref, wt_ref, a8_ref, wh_ref, s12_ref):
    wh = jnp.dot(x_ref[...], wt_ref[...], preferred_element_type=jnp.float32)
    wh_ref[...] = wh
    s12_ref[...] = jnp.dot(wh, a8_ref[...], preferred_element_type=jnp.float32)


def _proj(x, wt, a8):
    B = 1000
    return pl.pallas_call(
        _proj_body,
        grid=(N // B,),
        in_specs=[
            pl.BlockSpec((B, D), lambda i: (i, 0)),
            pl.BlockSpec((D, D), lambda i: (0, 0)),
            pl.BlockSpec((D, D), lambda i: (0, 0)),
        ],
        out_specs=[
            pl.BlockSpec((B, D), lambda i: (i, 0)),
            pl.BlockSpec((B, D), lambda i: (i, 0)),
        ],
        out_shape=[
            jax.ShapeDtypeStruct((N, D), jnp.float32),
            jax.ShapeDtypeStruct((N, D), jnp.float32),
        ],
    )(x, wt, a8)


def _wpass_body(src_hbm, dst_hbm, s1_hbm, s2_hbm,
                w_hbm, alpha_hbm,
                s1_v, s2_v, sv, dv, wv, alpha_acc, sem):
    c = lax.axis_index("c")
    s = lax.axis_index("s")
    wid = c * NS + s

    pltpu.sync_copy(s1_hbm, s1_v)
    pltpu.sync_copy(s2_hbm, s2_v)

    def _zw(i, carry):
        wv[pl.ds(i * 16, 16)] = jnp.zeros((16,), jnp.float32)
        return carry

    lax.fori_loop(0, CH1 // 16, _zw, 0)

    @pl.when(s == 0)
    def _():
        def _za(k, carry):
            pltpu.sync_copy(wv, alpha_acc.at[pl.ds(k * CH1, CH1)])
            return carry
        lax.fori_loop(0, N // CH1, _za, 0)

    plsc.subcore_barrier()

    def _chunk(k, carry):
        off = wid * EPW1 + k * CH1
        pltpu.sync_copy(src_hbm.at[pl.ds(off, CH1)], sv)
        pltpu.sync_copy(dst_hbm.at[pl.ds(off, CH1)], dv)

        def _w16(j, carry2):
            sidx = sv[pl.ds(j * 16, 16)]
            didx = dv[pl.ds(j * 16, 16)]
            e = (plsc.load_gather(s1_v, [sidx])
                 + plsc.load_gather(s2_v, [didx]))
            e = jnp.where(e > 0, e, 0.2 * e)
            wv[pl.ds(j * 16, 16)] = jnp.exp(e)
            return carry2

        lax.fori_loop(0, CH1 // 16, _w16, 0)

        pltpu.sync_copy(wv, w_hbm.at[pl.ds(off, CH1)])
        pltpu.async_copy(wv, alpha_acc.at[dv], sem, add=True).wait()
        return carry

    lax.fori_loop(0, NCH1, _chunk, 0)

    plsc.subcore_barrier()

    @pl.when(s == 0)
    def _():
        pltpu.sync_copy(alpha_acc, alpha_hbm.at[c, 0])


_wpass = pl.kernel(
    _wpass_body,
    out_type=(
        jax.ShapeDtypeStruct((E,), jnp.float32),
        jax.ShapeDtypeStruct((NC, 8, N), jnp.float32),
    ),
    mesh=plsc.VectorSubcoreMesh(core_axis_name="c", subcore_axis_name="s"),
    compiler_params=pltpu.CompilerParams(needs_layout_passes=False),
    scratch_types=[
        pltpu.VMEM((N,), jnp.float32),
        pltpu.VMEM((N,), jnp.float32),
        pltpu.VMEM((CH1,), jnp.int32),
        pltpu.VMEM((CH1,), jnp.int32),
        pltpu.VMEM((CH1,), jnp.float32),
        pltpu.VMEM_SHARED((N,), jnp.float32),
        pltpu.SemaphoreType.DMA,
    ],
)


def _edge_body(src_hbm, dst_hbm, w_hbm, wh_hbm,
               out_hbm,
               srcv0, srcv1, dstv0, dstv1, wv0, wv1, dstv20, dstv21,
               wm_v, rows0, rows1, srcvt, dstvt, wvt, dstv2t,
               out_acc, sem_g, sem_i, sem_s):
    srcv = (srcv0, srcv1)
    dstv = (dstv0, dstv1)
    wv = (wv0, wv1)
    dstv2 = (dstv20, dstv21)
    rows = (rows0, rows1)
    c = lax.axis_index("c")
    s = lax.axis_index("s")
    base = c * NPC

    def _zr(i, carry):
        for f in range(D // 16):
            rows0[i, pl.ds(f * 16, 16)] = jnp.zeros((16,), jnp.float32)
        return carry

    lax.fori_loop(0, CHUNK, _zr, 0)

    pltpu.sync_copy(rows0.at[pl.ds(0, RPT)],
                    out_acc.at[pl.ds(s * RPT, RPT)])

    @pl.when(s == NS - 1)
    def _():
        rem = ACC_ROWS - NS * RPT
        pltpu.sync_copy(rows0.at[pl.ds(0, rem)],
                        out_acc.at[pl.ds(NS * RPT, rem)])

    plsc.subcore_barrier()

    ebase = s * EPW

    pltpu.sync_copy(src_hbm.at[pl.ds(ebase, CHUNK)], srcv0)
    pltpu.sync_copy(dst_hbm.at[pl.ds(ebase, CHUNK)], dstv0)
    pltpu.sync_copy(w_hbm.at[pl.ds(ebase, CHUNK)], wv0)
    pltpu.async_copy(wh_hbm.at[srcv0], rows0, sem_g)

    def _pair(kk, carry):
        for b in range(2):
            nb = 1 - b
            k = kk * 2 + b
            have_next = (kk < NPAIR - 1) if b == 1 else True
            off_next = ebase + (k + 1) * CHUNK

            def _prefetch():
                pltpu.async_copy(src_hbm.at[pl.ds(off_next, CHUNK)],
                                 srcv[nb], sem_i)
                pltpu.async_copy(dst_hbm.at[pl.ds(off_next, CHUNK)],
                                 dstv[nb], sem_i)
                pltpu.async_copy(w_hbm.at[pl.ds(off_next, CHUNK)],
                                 wv[nb], sem_i)

            if b == 0:
                _prefetch()
            else:
                @pl.when(have_next)
                def _():
                    _prefetch()

            def _w16(j, carry2):
                didx = dstv[b][pl.ds(j * 16, 16)]
                w = wv[b][pl.ds(j * 16, 16)]
                lo = didx - base
                valid = jnp.logical_and(lo >= 0, lo < NPC)
                wm_v[pl.ds(j * 16, 16)] = jnp.where(valid, w, 0.0)
                dstv2[b][pl.ds(j * 16, 16)] = jnp.where(
                    valid, lo, jnp.bitwise_and(didx, 4095))
                return carry2

            lax.fori_loop(0, CHUNK // 16, _w16, 0)

            def _launch_next():
                pltpu.make_async_copy(src_hbm.at[pl.ds(off_next, CHUNK)],
                                      srcv[nb], sem_i).wait()
                pltpu.make_async_copy(dst_hbm.at[pl.ds(off_next, CHUNK)],
                                      dstv[nb], sem_i).wait()
                pltpu.make_async_copy(w_hbm.at[pl.ds(off_next, CHUNK)],
                                      wv[nb], sem_i).wait()
                pltpu.async_copy(wh_hbm.at[srcv[nb]], rows[nb], sem_g)

            def _wait_prev_scatter():
                pltpu.make_async_copy(rows[nb], out_acc.at[dstv2[nb]],
                                      sem_s).wait()

            if b == 0:
                @pl.when(kk >= 1)
                def _():
                    _wait_prev_scatter()
                _launch_next()
            else:
                @pl.when(have_next)
                def _():
                    _wait_prev_scatter()
                    _launch_next()

            pltpu.make_async_copy(wh_hbm.at[srcv[b]], rows[b],
                                  sem_g).wait()

            def _sc16(j, carry2):
                w = wm_v[pl.ds(j * 16, 16)]
                for l in range(16):
                    wl = w.at[jnp.full((16,), l, jnp.int32)].get(
                        mode="promise_in_bounds")
                    r = j * 16 + l
                    for f in range(D // 16):
                        rows[b][r, pl.ds(f * 16, 16)] = (
                            rows[b][r, pl.ds(f * 16, 16)] * wl)
                return carry2

            lax.fori_loop(0, CHUNK // 16, _sc16, 0)

            pltpu.async_copy(rows[b], out_acc.at[dstv2[b]], sem_s,
                             add=True)
        return carry

    lax.fori_loop(0, NPAIR, _pair, 0)

    offt = ebase + 2 * NPAIR * CHUNK
    pltpu.sync_copy(src_hbm.at[pl.ds(offt, TAIL)], srcvt)
    pltpu.sync_copy(dst_hbm.at[pl.ds(offt, TAIL)], dstvt)
    pltpu.sync_copy(w_hbm.at[pl.ds(offt, TAIL)], wvt)

    pltpu.make_async_copy(rows0, out_acc.at[dstv20], sem_s).wait()
    pltpu.async_copy(wh_hbm.at[srcvt], rows0.at[pl.ds(0, TAIL)], sem_g)

    def _w16t(j, carry2):
        didx = dstvt[pl.ds(j * 16, 16)]
        w = wvt[pl.ds(j * 16, 16)]
        lo = didx - base
        valid = jnp.logical_and(lo >= 0, lo < NPC)
        wm_v[pl.ds(j * 16, 16)] = jnp.where(valid, w, 0.0)
        dstv2t[pl.ds(j * 16, 16)] = jnp.where(
            valid, lo, jnp.bitwise_and(didx, 4095))
        return carry2

    lax.fori_loop(0, TAIL // 16, _w16t, 0)

    pltpu.make_async_copy(wh_hbm.at[srcvt], rows0.at[pl.ds(0, TAIL)],
                          sem_g).wait()

    def _sc16t(j, carry2):
        w = wm_v[pl.ds(j * 16, 16)]
        for l in range(16):
            wl = w.at[jnp.full((16,), l, jnp.int32)].get(
                mode="promise_in_bounds")
            r = j * 16 + l
            for f in range(D // 16):
                rows0[r, pl.ds(f * 16, 16)] = (
                    rows0[r, pl.ds(f * 16, 16)] * wl)
        return carry2

    lax.fori_loop(0, TAIL // 16, _sc16t, 0)

    pltpu.async_copy(rows0.at[pl.ds(0, TAIL)], out_acc.at[dstv2t], sem_s,
                     add=True).wait()
    pltpu.make_async_copy(rows1, out_acc.at[dstv21], sem_s).wait()

    plsc.subcore_barrier()

    pltpu.sync_copy(out_acc.at[pl.ds(s * RPT, RPT)],
                    out_hbm.at[c, pl.ds(s * RPT, RPT)])

    @pl.when(s == NS - 1)
    def _():
        rem = ACC_ROWS - NS * RPT
        pltpu.sync_copy(out_acc.at[pl.ds(NS * RPT, rem)],
                        out_hbm.at[c, pl.ds(NS * RPT, rem)])


_edge_kernel = pl.kernel(
    _edge_body,
    out_type=jax.ShapeDtypeStruct((NC, ACC_ROWS, D), jnp.float32),
    mesh=plsc.VectorSubcoreMesh(core_axis_name="c", subcore_axis_name="s"),
    compiler_params=pltpu.CompilerParams(needs_layout_passes=False),
    scratch_types=[
        pltpu.VMEM((CHUNK,), jnp.int32),
        pltpu.VMEM((CHUNK,), jnp.int32),
        pltpu.VMEM((CHUNK,), jnp.int32),
        pltpu.VMEM((CHUNK,), jnp.int32),
        pltpu.VMEM((CHUNK,), jnp.float32),
        pltpu.VMEM((CHUNK,), jnp.float32),
        pltpu.VMEM((CHUNK,), jnp.int32),
        pltpu.VMEM((CHUNK,), jnp.int32),
        pltpu.VMEM((CHUNK,), jnp.float32),
        pltpu.VMEM((CHUNK, D), jnp.float32),
        pltpu.VMEM((CHUNK, D), jnp.float32),
        pltpu.VMEM((TAIL,), jnp.int32),
        pltpu.VMEM((TAIL,), jnp.int32),
        pltpu.VMEM((TAIL,), jnp.float32),
        pltpu.VMEM((TAIL,), jnp.int32),
        pltpu.VMEM_SHARED((ACC_ROWS, D), jnp.float32),
        pltpu.SemaphoreType.DMA,
        pltpu.SemaphoreType.DMA,
        pltpu.SemaphoreType.DMA,
    ],
)


def _final_body(p_ref, at_ref, o_ref):
    a = jnp.maximum(at_ref[:, 0] + at_ref[:, 1], 1e-10)
    o_ref[...] = p_ref[...] / a[:, None]


def _final(out_cat, alpha_t):
    B = 1000
    return pl.pallas_call(
        _final_body,
        grid=(N // B,),
        in_specs=[
            pl.BlockSpec((B, D), lambda i: (i, 0)),
            pl.BlockSpec((B, 8), lambda i: (i, 0)),
        ],
        out_specs=pl.BlockSpec((B, D), lambda i: (i, 0)),
        out_shape=jax.ShapeDtypeStruct((N, D), jnp.float32),
    )(out_cat, alpha_t)


def kernel(x, edge_index, W, attn_w):
    src = edge_index[0].astype(jnp.int32)
    dst = edge_index[1].astype(jnp.int32)
    wt = W.T
    a1 = attn_w[0, :D]
    a2 = attn_w[0, D:]
    a8 = jnp.zeros((D, D), jnp.float32).at[:, 0].set(a1).at[:, 1].set(a2)
    wh, s12 = _proj(x, wt, a8)
    s1 = s12[:, 0]
    s2 = s12[:, 1]
    w_edges, alpha_parts = _wpass(src, dst, s1, s2)
    out_parts = _edge_kernel(src, dst, w_edges, wh)
    out_cat = jnp.concatenate(
        [out_parts[0, :NPC], out_parts[1, :NPC]], axis=0)
    ap = alpha_parts[:, 0, :]
    alpha_t = jnp.pad(ap.T, ((0, 0), (0, 6)))
    return _final(out_cat, alpha_t)


# speedup vs baseline: 1.5027x; 1.0493x over previous
import jax
import jax.numpy as jnp
from jax import lax
from jax.experimental import pallas as pl
from jax.experimental.pallas import tpu as pltpu
from jax.experimental.pallas import tpu_sc as plsc

N = 10000
E = 320000
D = 128

NC = 2
NS = 16
NW = NC * NS

EPW1 = E // NW
CH1 = 400
NCH1 = EPW1 // CH1

NPC = N // NC
ACC_ROWS = 5008
EPW = E // NS
CHUNK = 320
NPAIR = 31
TAIL = EPW - 2 * NPAIR * CHUNK
RPT = 312


def _proj_body(x_ref, wt_ref, a8_ref, wh_ref, s12_ref):
    wh = jnp.dot(x_ref[...], wt_ref[...], preferred_element_type=jnp.float32)
    wh_ref[...] = wh
    s12_ref[...] = jnp.dot(wh, a8_ref[...], preferred_element_type=jnp.float32)


def _proj(x, wt, a8):
    B = 1000
    return pl.pallas_call(
        _proj_body,
        grid=(N // B,),
        in_specs=[
            pl.BlockSpec((B, D), lambda i: (i, 0)),
            pl.BlockSpec((D, D), lambda i: (0, 0)),
            pl.BlockSpec((D, D), lambda i: (0, 0)),
        ],
        out_specs=[
            pl.BlockSpec((B, D), lambda i: (i, 0)),
            pl.BlockSpec((B, D), lambda i: (i, 0)),
        ],
        out_shape=[
            jax.ShapeDtypeStruct((N, D), jnp.float32),
            jax.ShapeDtypeStruct((N, D), jnp.float32),
        ],
    )(x, wt, a8)


def _wpass_body(src_hbm, dst_hbm, s1_hbm, s2_hbm,
                w_hbm, a0_hbm, a1_hbm,
                s1_v, s2_v, sv, dv, wv, alpha_acc, sem):
    c = lax.axis_index("c")
    s = lax.axis_index("s")
    wid = c * NS + s

    pltpu.sync_copy(s1_hbm, s1_v)
    pltpu.sync_copy(s2_hbm, s2_v)

    def _zw(i, carry):
        wv[pl.ds(i * 16, 16)] = jnp.zeros((16,), jnp.float32)
        return carry

    lax.fori_loop(0, CH1 // 16, _zw, 0)

    @pl.when(s == 0)
    def _():
        def _za(k, carry):
            pltpu.sync_copy(wv, alpha_acc.at[pl.ds(k * CH1, CH1)])
            return carry
        lax.fori_loop(0, N // CH1, _za, 0)

    plsc.subcore_barrier()

    def _chunk(k, carry):
        off = wid * EPW1 + k * CH1
        pltpu.sync_copy(src_hbm.at[pl.ds(off, CH1)], sv)
        pltpu.sync_copy(dst_hbm.at[pl.ds(off, CH1)], dv)

        def _w16(j, carry2):
            sidx = sv[pl.ds(j * 16, 16)]
            didx = dv[pl.ds(j * 16, 16)]
            e = (plsc.load_gather(s1_v, [sidx])
                 + plsc.load_gather(s2_v, [didx]))
            e = jnp.where(e > 0, e, 0.2 * e)
            wv[pl.ds(j * 16, 16)] = jnp.exp(e)
            return carry2

        lax.fori_loop(0, CH1 // 16, _w16, 0)

        pltpu.sync_copy(wv, w_hbm.at[pl.ds(off, CH1)])
        pltpu.async_copy(wv, alpha_acc.at[dv], sem, add=True).wait()
        return carry

    lax.fori_loop(0, NCH1, _chunk, 0)

    plsc.subcore_barrier()

    @pl.when(jnp.logical_and(c == 0, s == 0))
    def _():
        pltpu.sync_copy(alpha_acc, a0_hbm)

    @pl.when(jnp.logical_and(c == 1, s == 0))
    def _():
        pltpu.sync_copy(alpha_acc, a1_hbm)


_wpass = pl.kernel(
    _wpass_body,
    out_type=(
        jax.ShapeDtypeStruct((E,), jnp.float32),
        jax.ShapeDtypeStruct((N,), jnp.float32),
        jax.ShapeDtypeStruct((N,), jnp.float32),
    ),
    mesh=plsc.VectorSubcoreMesh(core_axis_name="c", subcore_axis_name="s"),
    compiler_params=pltpu.CompilerParams(needs_layout_passes=False),
    scratch_types=[
        pltpu.VMEM((N,), jnp.float32),
        pltpu.VMEM((N,), jnp.float32),
        pltpu.VMEM((CH1,), jnp.int32),
        pltpu.VMEM((CH1,), jnp.int32),
        pltpu.VMEM((CH1,), jnp.float32),
        pltpu.VMEM_SHARED((N,), jnp.float32),
        pltpu.SemaphoreType.DMA,
    ],
)


def _edge_body(src_hbm, dst_hbm, w_hbm, wh_hbm, a0_hbm, a1_hbm,
               out_hbm,
               srcv0, srcv1, dstv0, dstv1, wv0, wv1, dstv20, dstv21,
               wm_v, rows0, rows1, srcvt, dstvt, wvt, dstv2t, av0, av1,
               out_acc, sem_g, sem_i, sem_s):
    srcv = (srcv0, srcv1)
    dstv = (dstv0, dstv1)
    wv = (wv0, wv1)
    dstv2 = (dstv20, dstv21)
    rows = (rows0, rows1)
    c = lax.axis_index("c")
    s = lax.axis_index("s")
    base = c * NPC

    def _zr(i, carry):
        for f in range(D // 16):
            rows0[i, pl.ds(f * 16, 16)] = jnp.zeros((16,), jnp.float32)
        return carry

    lax.fori_loop(0, CHUNK, _zr, 0)

    pltpu.sync_copy(rows0.at[pl.ds(0, RPT)],
                    out_acc.at[pl.ds(s * RPT, RPT)])

    @pl.when(s == NS - 1)
    def _():
        rem = ACC_ROWS - NS * RPT
        pltpu.sync_copy(rows0.at[pl.ds(0, rem)],
                        out_acc.at[pl.ds(NS * RPT, rem)])

    plsc.subcore_barrier()

    ebase = s * EPW

    pltpu.sync_copy(src_hbm.at[pl.ds(ebase, CHUNK)], srcv0)
    pltpu.sync_copy(dst_hbm.at[pl.ds(ebase, CHUNK)], dstv0)
    pltpu.sync_copy(w_hbm.at[pl.ds(ebase, CHUNK)], wv0)
    pltpu.async_copy(wh_hbm.at[srcv0], rows0, sem_g)

    def _pair(kk, carry):
        for b in range(2):
            nb = 1 - b
            k = kk * 2 + b
            have_next = (kk < NPAIR - 1) if b == 1 else True
            off_next = ebase + (k + 1) * CHUNK

            def _prefetch():
                pltpu.async_copy(src_hbm.at[pl.ds(off_next, CHUNK)],
                                 srcv[nb], sem_i)
                pltpu.async_copy(dst_hbm.at[pl.ds(off_next, CHUNK)],
                                 dstv[nb], sem_i)
                pltpu.async_copy(w_hbm.at[pl.ds(off_next, CHUNK)],
                                 wv[nb], sem_i)

            if b == 0:
                _prefetch()
            else:
                @pl.when(have_next)
                def _():
                    _prefetch()

            def _w16(j, carry2):
                didx = dstv[b][pl.ds(j * 16, 16)]
                w = wv[b][pl.ds(j * 16, 16)]
                lo = didx - base
                valid = jnp.logical_and(lo >= 0, lo < NPC)
                wm_v[pl.ds(j * 16, 16)] = jnp.where(valid, w, 0.0)
                dstv2[b][pl.ds(j * 16, 16)] = jnp.where(
                    valid, lo, jnp.bitwise_and(didx, 4095))
                return carry2

            lax.fori_loop(0, CHUNK // 16, _w16, 0)

            def _launch_next():
                pltpu.make_async_copy(src_hbm.at[pl.ds(off_next, CHUNK)],
                                      srcv[nb], sem_i).wait()
                pltpu.make_async_copy(dst_hbm.at[pl.ds(off_next, CHUNK)],
                                      dstv[nb], sem_i).wait()
                pltpu.make_async_copy(w_hbm.at[pl.ds(off_next, CHUNK)],
                                      wv[nb], sem_i).wait()
                pltpu.async_copy(wh_hbm.at[srcv[nb]], rows[nb], sem_g)

            def _wait_prev_scatter():
                pltpu.make_async_copy(rows[nb], out_acc.at[dstv2[nb]],
                                      sem_s).wait()

            if b == 0:
                @pl.when(kk >= 1)
                def _():
                    _wait_prev_scatter()
                _launch_next()
            else:
                @pl.when(have_next)
                def _():
                    _wait_prev_scatter()
                    _launch_next()

            pltpu.make_async_copy(wh_hbm.at[srcv[b]], rows[b],
                                  sem_g).wait()

            def _sc16(j, carry2):
                w = wm_v[pl.ds(j * 16, 16)]
                for l in range(16):
                    wl = w.at[jnp.full((16,), l, jnp.int32)].get(
                        mode="promise_in_bounds")
                    r = j * 16 + l
                    for f in range(D // 16):
                        rows[b][r, pl.ds(f * 16, 16)] = (
                            rows[b][r, pl.ds(f * 16, 16)] * wl)
                return carry2

            lax.fori_loop(0, CHUNK // 16, _sc16, 0)

            pltpu.async_copy(rows[b], out_acc.at[dstv2[b]], sem_s,
                             add=True)
        return carry

    lax.fori_loop(0, NPAIR, _pair, 0)

    offt = ebase + 2 * NPAIR * CHUNK
    pltpu.sync_copy(src_hbm.at[pl.ds(offt, TAIL)], srcvt)
    pltpu.sync_copy(dst_hbm.at[pl.ds(offt, TAIL)], dstvt)
    pltpu.sync_copy(w_hbm.at[pl.ds(offt, TAIL)], wvt)

    pltpu.make_async_copy(rows0, out_acc.at[dstv20], sem_s).wait()
    pltpu.async_copy(wh_hbm.at[srcvt], rows0.at[pl.ds(0, TAIL)], sem_g)

    def _w16t(j, carry2):
        didx = dstvt[pl.ds(j * 16, 16)]
        w = wvt[pl.ds(j * 16, 16)]
        lo = didx - base
        valid = jnp.logical_and(lo >= 0, lo < NPC)
        wm_v[pl.ds(j * 16, 16)] = jnp.where(valid, w, 0.0)
        dstv2t[pl.ds(j * 16, 16)] = jnp.where(
            valid, lo, jnp.bitwise_and(didx, 4095))
        return carry2

    lax.fori_loop(0, TAIL // 16, _w16t, 0)

    pltpu.make_async_copy(wh_hbm.at[srcvt], rows0.at[pl.ds(0, TAIL)],
                          sem_g).wait()

    def _sc16t(j, carry2):
        w = wm_v[pl.ds(j * 16, 16)]
        for l in range(16):
            wl = w.at[jnp.full((16,), l, jnp.int32)].get(
                mode="promise_in_bounds")
            r = j * 16 + l
            for f in range(D // 16):
                rows0[r, pl.ds(f * 16, 16)] = (
                    rows0[r, pl.ds(f * 16, 16)] * wl)
        return carry2

    lax.fori_loop(0, TAIL // 16, _sc16t, 0)

    pltpu.async_copy(rows0.at[pl.ds(0, TAIL)], out_acc.at[dstv2t], sem_s,
                     add=True).wait()
    pltpu.make_async_copy(rows1, out_acc.at[dstv21], sem_s).wait()

    plsc.subcore_barrier()

    CPR = 320
    lrow = s * RPT
    grow = c * NPC + lrow
    pltpu.sync_copy(out_acc.at[pl.ds(lrow, CPR)], rows0)
    pltpu.sync_copy(a0_hbm.at[pl.ds(grow, CPR)], av0)
    pltpu.sync_copy(a1_hbm.at[pl.ds(grow, CPR)], av1)

    def _div16(j, carry):
        a16 = av0[pl.ds(j * 16, 16)] + av1[pl.ds(j * 16, 16)]
        inv = 1.0 / jnp.maximum(a16, 1e-10)
        for l in range(16):
            il = inv.at[jnp.full((16,), l, jnp.int32)].get(
                mode="promise_in_bounds")
            r = j * 16 + l
            for f in range(D // 16):
                rows1[r, pl.ds(f * 16, 16)] = (
                    rows0[r, pl.ds(f * 16, 16)] * il)
        return carry

    lax.fori_loop(0, CPR // 16, _div16, 0)

    pltpu.sync_copy(rows1, out_hbm.at[pl.ds(grow, CPR)])


_edge_kernel = pl.kernel(
    _edge_body,
    out_type=jax.ShapeDtypeStruct((N, D), jnp.float32),
    mesh=plsc.VectorSubcoreMesh(core_axis_name="c", subcore_axis_name="s"),
    compiler_params=pltpu.CompilerParams(needs_layout_passes=False),
    scratch_types=[
        pltpu.VMEM((CHUNK,), jnp.int32),
        pltpu.VMEM((CHUNK,), jnp.int32),
        pltpu.VMEM((CHUNK,), jnp.int32),
        pltpu.VMEM((CHUNK,), jnp.int32),
        pltpu.VMEM((CHUNK,), jnp.float32),
        pltpu.VMEM((CHUNK,), jnp.float32),
        pltpu.VMEM((CHUNK,), jnp.int32),
        pltpu.VMEM((CHUNK,), jnp.int32),
        pltpu.VMEM((CHUNK,), jnp.float32),
        pltpu.VMEM((CHUNK, D), jnp.float32),
        pltpu.VMEM((CHUNK, D), jnp.float32),
        pltpu.VMEM((TAIL,), jnp.int32),
        pltpu.VMEM((TAIL,), jnp.int32),
        pltpu.VMEM((TAIL,), jnp.float32),
        pltpu.VMEM((TAIL,), jnp.int32),
        pltpu.VMEM((320,), jnp.float32),
        pltpu.VMEM((320,), jnp.float32),
        pltpu.VMEM_SHARED((ACC_ROWS, D), jnp.float32),
        pltpu.SemaphoreType.DMA,
        pltpu.SemaphoreType.DMA,
        pltpu.SemaphoreType.DMA,
    ],
)


def kernel(x, edge_index, W, attn_w):
    src = edge_index[0].astype(jnp.int32)
    dst = edge_index[1].astype(jnp.int32)
    wt = W.T
    a1 = attn_w[0, :D]
    a2 = attn_w[0, D:]
    a8 = jnp.zeros((D, D), jnp.float32).at[:, 0].set(a1).at[:, 1].set(a2)
    wh, s12 = _proj(x, wt, a8)
    s1 = s12[:, 0]
    s2 = s12[:, 1]
    w_edges, a0, a1 = _wpass(src, dst, s1, s2)
    return _edge_kernel(src, dst, w_edges, wh, a0, a1)
